# baseline (speedup 1.0000x reference)
import jax
import jax.numpy as jnp
from jax.experimental import pallas as pl

N = 10000
E = 160000
B = 8
T = 4
F_IN = 1
U = 32


def _identity_kernel(x_ref, o_ref):
    o_ref[...] = x_ref[...]


def kernel(x, edge_index, W_gat, att_src, att_dst, b_gat, W1, b1, W2, b2):
    loop = jnp.arange(N, dtype=edge_index.dtype)
    src = jnp.concatenate([edge_index[0], loop])
    dst = jnp.concatenate([edge_index[1], loop])

    def gat(xn):
        h = xn @ W_gat
        a_s = h @ att_src
        a_d = h @ att_dst
        alpha = jax.nn.leaky_relu(a_s[src] + a_d[dst], negative_slope=0.2)
        m = jax.ops.segment_max(alpha, dst, num_segments=N)
        m = jnp.where(jnp.isfinite(m), m, 0.0)
        e = jnp.exp(alpha - jax.lax.stop_gradient(m)[dst])
        denom = jax.ops.segment_sum(e, dst, num_segments=N)
        num = jax.ops.segment_sum(e[:, None] * h[src], dst, num_segments=N)
        return num / jnp.maximum(denom, 1e-16)[:, None] + b_gat

    def cell(state, inp_t):
        xs = jnp.concatenate([state, inp_t], axis=2)
        s2 = jax.vmap(gat)(xs)
        cat1 = jnp.concatenate([inp_t, s2], axis=2)
        v = jax.nn.sigmoid(cat1 @ W1 + b1)
        r, u = v[..., :U], v[..., U:]
        cat2 = jnp.concatenate([inp_t, r * s2], axis=2)
        c = jnp.tanh(cat2 @ W2 + b2)
        return u * s2 + (1.0 - u) * c

    state = jnp.zeros((B, N, U), dtype=jnp.float32)
    for t in range(T):
        state = cell(state, x[:, t])
    out = state.reshape(B, N * U)
    return pl.pallas_call(
        _identity_kernel,
        out_shape=jax.ShapeDtypeStruct((B, N * U), jnp.float32),
    )(out)


# speedup vs baseline: 9.3605x; 9.3605x over previous
import functools

import jax
import jax.numpy as jnp
from jax import lax
from jax.experimental import pallas as pl
from jax.experimental.pallas import tpu as pltpu
from jax.experimental.pallas import tpu_sc as plsc

N = 10000
E = 160000
B = 8
T = 4
U = 32

NS = 16
NC = 2
W = 512
EP = E + N
EPT = ((EP + NS * W - 1) // (NS * W)) * W
E_PAD = EPT * NS
NWIN = EPT // W
PAD_ROWS = 8
RC = 640
RC_LAST = N - RC * (NS - 1)
NP = RC * NS
BPC = B // NC


def _tc_pre_body(state_ref, xt_ref, wgh_ref, wgx_ref, ats_ref, atd_ref,
                 h_ref, as_ref, ad_ref, amax_ref):
    st = state_ref[0]
    xt = xt_ref[0]
    h = jnp.dot(st, wgh_ref[...], preferred_element_type=jnp.float32)
    h = h + xt * wgx_ref[...]
    h_ref[0] = h
    a_s = jnp.sum(h * ats_ref[...], axis=1, keepdims=True)
    a_d = jnp.sum(h * atd_ref[...], axis=1, keepdims=True)
    as_ref[0] = a_s
    ad_ref[0] = a_d
    amax_ref[0] = jnp.full((1, 16), jnp.max(a_s), dtype=jnp.float32)


def _tc_pre(state, xt, wgh, wgx, ats, atd):
    return pl.pallas_call(
        _tc_pre_body,
        grid=(B,),
        in_specs=[
            pl.BlockSpec((1, N, U), lambda b: (b, 0, 0)),
            pl.BlockSpec((1, N, 1), lambda b: (b, 0, 0)),
            pl.BlockSpec((U, U), lambda b: (0, 0)),
            pl.BlockSpec((1, U), lambda b: (0, 0)),
            pl.BlockSpec((1, U), lambda b: (0, 0)),
            pl.BlockSpec((1, U), lambda b: (0, 0)),
        ],
        out_specs=[
            pl.BlockSpec((1, N, U), lambda b: (b, 0, 0)),
            pl.BlockSpec((1, N, 1), lambda b: (b, 0, 0)),
            pl.BlockSpec((1, N, 1), lambda b: (b, 0, 0)),
            pl.BlockSpec((1, 1, 16), lambda b: (b, 0, 0)),
        ],
        out_shape=[
            jax.ShapeDtypeStruct((B, N, U), jnp.float32),
            jax.ShapeDtypeStruct((B, N, 1), jnp.float32),
            jax.ShapeDtypeStruct((B, N, 1), jnp.float32),
            jax.ShapeDtypeStruct((B, 1, 16), jnp.float32),
        ],
    )(state, xt, wgh, wgx, ats, atd)


def _tc_post_body(num_ref, den_ref, xt_ref, bg_ref,
                  w1x_ref, w1h_ref, b1_ref, w2x_ref, w2h_ref, b2_ref,
                  out_ref):
    num = num_ref[0]
    den = den_ref[0]
    xt = xt_ref[0]
    s2 = num / jnp.maximum(den, 1e-30) + bg_ref[...]
    v = xt * w1x_ref[...] + jnp.dot(s2, w1h_ref[...],
                                    preferred_element_type=jnp.float32)
    v = jax.nn.sigmoid(v + b1_ref[...])
    r = v[:, :U]
    u = v[:, U:]
    c = xt * w2x_ref[...] + jnp.dot(r * s2, w2h_ref[...],
                                    preferred_element_type=jnp.float32)
    c = jnp.tanh(c + b2_ref[...])
    out_ref[0] = u * s2 + (1.0 - u) * c


def _tc_post(num, den, xt, bg, w1x, w1h, b1, w2x, w2h, b2):
    return pl.pallas_call(
        _tc_post_body,
        grid=(B,),
        in_specs=[
            pl.BlockSpec((1, N, U), lambda b: (b, 0, 0)),
            pl.BlockSpec((1, N, 1), lambda b: (b, 0, 0)),
            pl.BlockSpec((1, N, 1), lambda b: (b, 0, 0)),
            pl.BlockSpec((1, U), lambda b: (0, 0)),
            pl.BlockSpec((1, 2 * U), lambda b: (0, 0)),
            pl.BlockSpec((U, 2 * U), lambda b: (0, 0)),
            pl.BlockSpec((1, 2 * U), lambda b: (0, 0)),
            pl.BlockSpec((1, U), lambda b: (0, 0)),
            pl.BlockSpec((U, U), lambda b: (0, 0)),
            pl.BlockSpec((1, U), lambda b: (0, 0)),
        ],
        out_specs=pl.BlockSpec((1, N, U), lambda b: (b, 0, 0)),
        out_shape=jax.ShapeDtypeStruct((B, N, U), jnp.float32),
    )(num, den, xt, bg, w1x, w1h, b1, w2x, w2h, b2)


def _sc_edge_body(h_hbm, as_hbm, ad_hbm, amax_hbm, src_hbm, dst_hbm,
                  num_out, den_out,
                  h_sp, tnum_sp, tden_sp,
                  as_t, ad_t, amax_t, src_w, dst_w, rows, ev_buf,
                  znum, zden):
    c = lax.axis_index("c")
    s = lax.axis_index("s")
    zero16 = jnp.zeros((16,), jnp.float32)
    e16 = lax.iota(jnp.int32, 16)

    def _zn(i, carry):
        znum[i, pl.ds(0, 16)] = zero16
        znum[i, pl.ds(16, 16)] = zero16
        return carry
    lax.fori_loop(0, RC, _zn, 0)

    def _zd(i, carry):
        zden[pl.ds(i * 16, 16)] = zero16
        return carry
    lax.fori_loop(0, RC // 16, _zd, 0)

    row0 = s * RC
    is_last = s == NS - 1

    def batch_body(i, carry):
        b = c * BPC + i
        def stage_rows(sz):
            def f():
                pltpu.sync_copy(znum.at[pl.ds(0, sz)],
                                tnum_sp.at[pl.ds(row0, sz)])
                pltpu.sync_copy(h_hbm.at[b, pl.ds(row0, sz)],
                                h_sp.at[pl.ds(row0, sz)])
            return f
        pl.when(is_last)(stage_rows(RC_LAST))
        pl.when(jnp.logical_not(is_last))(stage_rows(RC))
        pltpu.sync_copy(zden, tden_sp.at[pl.ds(row0, RC)])
        pltpu.sync_copy(as_hbm.at[b, 0], as_t)
        pltpu.sync_copy(ad_hbm.at[b, 0], ad_t)
        pltpu.sync_copy(amax_hbm.at[b], amax_t)
        plsc.subcore_barrier()

        amax_v = amax_t[0]

        def win_body(w, carry2):
            base = s * EPT + w * W
            pltpu.sync_copy(src_hbm.at[pl.ds(base, W)], src_w)
            pltpu.sync_copy(dst_hbm.at[pl.ds(base, W)], dst_w)
            pltpu.sync_copy(h_sp.at[src_w], rows)

            def j_body(j, carry3):
                jb = j * 16
                sv = src_w[pl.ds(jb, 16)]
                dv = dst_w[pl.ds(jb, 16)]
                asv = plsc.load_gather(as_t, [sv])
                adv = plsc.load_gather(ad_t, [dv])
                al = asv + adv
                al = jnp.where(al >= 0.0, al, al * 0.2)
                mv = amax_v + adv
                mv = jnp.where(mv >= 0.0, mv, mv * 0.2)
                ev = jnp.exp(al - mv)
                ev_buf[pl.ds(jb, 16)] = ev
                ridx = e16 + jb
                for u in range(U):
                    uv = jnp.full((16,), u, jnp.int32)
                    g = plsc.load_gather(rows, [ridx, uv])
                    plsc.store_scatter(rows, [ridx, uv], g * ev)
                return carry3
            lax.fori_loop(0, W // 16, j_body, 0)

            pltpu.sync_copy(rows, tnum_sp.at[dst_w], add=True)
            pltpu.sync_copy(ev_buf, tden_sp.at[dst_w], add=True)
            return carry2
        lax.fori_loop(0, NWIN, win_body, 0)
        plsc.subcore_barrier()

        def readout(sz):
            def f():
                pltpu.sync_copy(tnum_sp.at[pl.ds(row0, sz)],
                                num_out.at[b, pl.ds(row0, sz)])
            return f
        pl.when(is_last)(readout(RC_LAST))
        pl.when(jnp.logical_not(is_last))(readout(RC))
        pltpu.sync_copy(tden_sp.at[pl.ds(row0, RC)],
                        den_out.at[b, 0, pl.ds(row0, RC)])
        plsc.subcore_barrier()
        return carry
    lax.fori_loop(0, BPC, batch_body, 0)


def _sc_edge(h, a_s, a_d, amax, src, dst):
    mesh = plsc.VectorSubcoreMesh(core_axis_name="c", subcore_axis_name="s")
    f = functools.partial(
        pl.kernel, _sc_edge_body, mesh=mesh,
        compiler_params=pltpu.CompilerParams(needs_layout_passes=False,
                                             use_tc_tiling_on_sc=False),
        out_type=[
            jax.ShapeDtypeStruct((B, N, U), jnp.float32),
            jax.ShapeDtypeStruct((B, 1, NP), jnp.float32),
        ],
        scratch_types=[
            pltpu.VMEM_SHARED((N, U), jnp.float32),
            pltpu.VMEM_SHARED((N + PAD_ROWS, U), jnp.float32),
            pltpu.VMEM_SHARED((NP,), jnp.float32),
            pltpu.VMEM((NP,), jnp.float32),
            pltpu.VMEM((NP,), jnp.float32),
            pltpu.VMEM((1, 16), jnp.float32),
            pltpu.VMEM((W,), jnp.int32),
            pltpu.VMEM((W,), jnp.int32),
            pltpu.VMEM((W, U), jnp.float32),
            pltpu.VMEM((W,), jnp.float32),
            pltpu.VMEM((RC, U), jnp.float32),
            pltpu.VMEM((RC,), jnp.float32),
        ],
    )()
    return f(h, a_s, a_d, amax, src, dst)


def kernel(x, edge_index, W_gat, att_src, att_dst, b_gat, W1, b1, W2, b2):
    loop = jnp.arange(N, dtype=jnp.int32)
    pad = E_PAD - EP
    src = jnp.concatenate([edge_index[0], loop,
                           jnp.zeros((pad,), jnp.int32)])
    dst = jnp.concatenate([edge_index[1], loop,
                           N + (jnp.arange(pad, dtype=jnp.int32) % PAD_ROWS)])

    wgh = W_gat[:U]
    wgx = W_gat[U:U + 1]
    ats = att_src.reshape(1, U)
    atd = att_dst.reshape(1, U)
    bg = b_gat.reshape(1, U)
    w1x = W1[0:1]
    w1h = W1[1:]
    b1r = b1.reshape(1, 2 * U)
    w2x = W2[0:1]
    w2h = W2[1:]
    b2r = b2.reshape(1, U)

    state = jnp.zeros((B, N, U), dtype=jnp.float32)
    for t in range(T):
        xt = x[:, t]
        h, a_s, a_d, amax = _tc_pre(state, xt, wgh, wgx, ats, atd)
        pad3 = ((0, 0), (0, 0), (0, NP - N))
        num, den = _sc_edge(h,
                            jnp.pad(a_s.reshape(B, 1, N), pad3),
                            jnp.pad(a_d.reshape(B, 1, N), pad3),
                            amax, src, dst)
        state = _tc_post(num, den[:, 0, :N].reshape(B, N, 1), xt, bg,
                         w1x, w1h, b1r, w2x, w2h, b2r)
    return state.reshape(B, N * U)


# speedup vs baseline: 31.2935x; 3.3431x over previous
import functools

import jax
import jax.numpy as jnp
from jax import lax
from jax.experimental import pallas as pl
from jax.experimental.pallas import tpu as pltpu
from jax.experimental.pallas import tpu_sc as plsc

N = 10000
E = 160000
B = 8
T = 4
U = 32

NS = 16
NC = 2
W = 512
EP = E + N
EPT = ((EP + NS * W - 1) // (NS * W)) * W
E_PAD = EPT * NS
NWIN = EPT // W
PAD_ROWS = 8
RC = 640
RC_LAST = N - RC * (NS - 1)
NP = RC * NS
ZC = 80
BPC = B // NC


def _tc_pre_body(state_ref, xt_ref, wgh_ref, wgx_ref, ats_ref, atd_ref,
                 h_ref, as_ref, ad_ref, amax_ref):
    st = state_ref[0]
    xt = xt_ref[0]
    h = jnp.dot(st, wgh_ref[...], preferred_element_type=jnp.float32)
    h = h + xt * wgx_ref[...]
    h_ref[0] = h
    a_s = jnp.sum(h * ats_ref[...], axis=1, keepdims=True)
    a_d = jnp.sum(h * atd_ref[...], axis=1, keepdims=True)
    as_ref[0] = a_s
    ad_ref[0] = a_d
    amax_ref[0] = jnp.full((1, 16), jnp.max(a_s), dtype=jnp.float32)


def _tc_pre(state, xt, wgh, wgx, ats, atd):
    return pl.pallas_call(
        _tc_pre_body,
        grid=(B,),
        in_specs=[
            pl.BlockSpec((1, N, U), lambda b: (b, 0, 0)),
            pl.BlockSpec((1, N, 1), lambda b: (b, 0, 0)),
            pl.BlockSpec((U, U), lambda b: (0, 0)),
            pl.BlockSpec((1, U), lambda b: (0, 0)),
            pl.BlockSpec((1, U), lambda b: (0, 0)),
            pl.BlockSpec((1, U), lambda b: (0, 0)),
        ],
        out_specs=[
            pl.BlockSpec((1, N, U), lambda b: (b, 0, 0)),
            pl.BlockSpec((1, N, 1), lambda b: (b, 0, 0)),
            pl.BlockSpec((1, N, 1), lambda b: (b, 0, 0)),
            pl.BlockSpec((1, 1, 16), lambda b: (b, 0, 0)),
        ],
        out_shape=[
            jax.ShapeDtypeStruct((B, N, U), jnp.float32),
            jax.ShapeDtypeStruct((B, N, 1), jnp.float32),
            jax.ShapeDtypeStruct((B, N, 1), jnp.float32),
            jax.ShapeDtypeStruct((B, 1, 16), jnp.float32),
        ],
    )(state, xt, wgh, wgx, ats, atd)


def _tc_post_body(num_ref, den_ref, xt_ref, bg_ref,
                  w1x_ref, w1h_ref, b1_ref, w2x_ref, w2h_ref, b2_ref,
                  out_ref):
    num = num_ref[0]
    den = den_ref[0]
    xt = xt_ref[0]
    s2 = num / jnp.maximum(den, 1e-30) + bg_ref[...]
    v = xt * w1x_ref[...] + jnp.dot(s2, w1h_ref[...],
                                    preferred_element_type=jnp.float32)
    v = jax.nn.sigmoid(v + b1_ref[...])
    r = v[:, :U]
    u = v[:, U:]
    c = xt * w2x_ref[...] + jnp.dot(r * s2, w2h_ref[...],
                                    preferred_element_type=jnp.float32)
    c = jnp.tanh(c + b2_ref[...])
    out_ref[0] = u * s2 + (1.0 - u) * c


def _tc_post(num, den, xt, bg, w1x, w1h, b1, w2x, w2h, b2):
    return pl.pallas_call(
        _tc_post_body,
        grid=(B,),
        in_specs=[
            pl.BlockSpec((1, N, U), lambda b: (b, 0, 0)),
            pl.BlockSpec((1, N, 1), lambda b: (b, 0, 0)),
            pl.BlockSpec((1, N, 1), lambda b: (b, 0, 0)),
            pl.BlockSpec((1, U), lambda b: (0, 0)),
            pl.BlockSpec((1, 2 * U), lambda b: (0, 0)),
            pl.BlockSpec((U, 2 * U), lambda b: (0, 0)),
            pl.BlockSpec((1, 2 * U), lambda b: (0, 0)),
            pl.BlockSpec((1, U), lambda b: (0, 0)),
            pl.BlockSpec((U, U), lambda b: (0, 0)),
            pl.BlockSpec((1, U), lambda b: (0, 0)),
        ],
        out_specs=pl.BlockSpec((1, N, U), lambda b: (b, 0, 0)),
        out_shape=jax.ShapeDtypeStruct((B, N, U), jnp.float32),
    )(num, den, xt, bg, w1x, w1h, b1, w2x, w2h, b2)


def _sc_edge_body(h_hbm, as_hbm, ad_hbm, amax_hbm, src_hbm, dst_hbm,
                  num_out, den_out,
                  h_sp, tnum_sp, tden_sp,
                  as_t, ad_t, amax_t, src_w, dst_w, rows, ev_buf,
                  znum, zden,
                  semi0, semi1, semi2, semg0, semg1, semg2,
                  sems0, sems1, sems2):
    sem_i = [semi0, semi1, semi2]
    sem_g = [semg0, semg1, semg2]
    sem_s = [sems0, sems1, sems2]
    c = lax.axis_index("c")
    s = lax.axis_index("s")
    zero16 = jnp.zeros((16,), jnp.float32)
    e16 = lax.iota(jnp.int32, 16)

    def _zn(i, carry):
        znum[i, pl.ds(0, 16)] = zero16
        znum[i, pl.ds(16, 16)] = zero16
        return carry
    lax.fori_loop(0, ZC, _zn, 0)

    def _zd(i, carry):
        zden[pl.ds(i * 16, 16)] = zero16
        return carry
    lax.fori_loop(0, RC // 16, _zd, 0)

    row0 = s * RC
    is_last = s == NS - 1

    def batch_body(i, carry):
        b = c * BPC + i
        def stage_rows(sz):
            def f():
                for i in range(sz // ZC):
                    pltpu.sync_copy(znum,
                                    tnum_sp.at[pl.ds(row0 + i * ZC, ZC)])
                pltpu.sync_copy(h_hbm.at[b, pl.ds(row0, sz)],
                                h_sp.at[pl.ds(row0, sz)])
            return f
        pl.when(is_last)(stage_rows(RC_LAST))
        pl.when(jnp.logical_not(is_last))(stage_rows(RC))
        pltpu.sync_copy(zden, tden_sp.at[pl.ds(row0, RC)])
        pltpu.sync_copy(as_hbm.at[b, 0], as_t)
        pltpu.sync_copy(ad_hbm.at[b, 0], ad_t)
        pltpu.sync_copy(amax_hbm.at[b], amax_t)
        plsc.subcore_barrier()

        amax_v = amax_t[0]

        tb = s * EPT

        def compute_win(p):
            rw = rows.at[p]
            def j_body(j, carry3):
                jb = j * 16
                sv = src_w[p, pl.ds(jb, 16)]
                dv = dst_w[p, pl.ds(jb, 16)]
                asv = plsc.load_gather(as_t, [sv])
                adv = plsc.load_gather(ad_t, [dv])
                al = asv + adv
                al = jnp.where(al >= 0.0, al, al * 0.2)
                mv = amax_v + adv
                mv = jnp.where(mv >= 0.0, mv, mv * 0.2)
                ev = jnp.exp(al - mv)
                ev_buf[p, pl.ds(jb, 16)] = ev
                for i in range(16):
                    evv = jnp.full((16,), ev[i], jnp.float32)
                    r0 = rw[jb + i, pl.ds(0, 16)]
                    rw[jb + i, pl.ds(0, 16)] = r0 * evv
                    r1 = rw[jb + i, pl.ds(16, 16)]
                    rw[jb + i, pl.ds(16, 16)] = r1 * evv
                return carry3
            lax.fori_loop(0, W // 16, j_body, 0)

        def win_body(w, carry2):
            base = tb + w * W
            p = 0
            pltpu.sync_copy(src_hbm.at[pl.ds(base, W)], src_w.at[p])
            pltpu.sync_copy(dst_hbm.at[pl.ds(base, W)], dst_w.at[p])
            pltpu.sync_copy(h_sp.at[src_w.at[p]], rows.at[p])
            compute_win(p)
            pltpu.sync_copy(rows.at[p], tnum_sp.at[dst_w.at[p]], add=True)
            pltpu.sync_copy(ev_buf.at[p], tden_sp.at[dst_w.at[p]], add=True)
            return carry2
        lax.fori_loop(0, NWIN, win_body, 0)
        plsc.subcore_barrier()

        def readout(sz):
            def f():
                pltpu.sync_copy(tnum_sp.at[pl.ds(row0, sz)],
                                num_out.at[b, pl.ds(row0, sz)])
            return f
        pl.when(is_last)(readout(RC_LAST))
        pl.when(jnp.logical_not(is_last))(readout(RC))
        pltpu.sync_copy(tden_sp.at[pl.ds(row0, RC)],
                        den_out.at[b, 0, pl.ds(row0, RC)])
        plsc.subcore_barrier()
        return carry
    lax.fori_loop(0, BPC, batch_body, 0)


def _sc_edge(h, a_s, a_d, amax, src, dst):
    mesh = plsc.VectorSubcoreMesh(core_axis_name="c", subcore_axis_name="s")
    f = functools.partial(
        pl.kernel, _sc_edge_body, mesh=mesh,
        compiler_params=pltpu.CompilerParams(needs_layout_passes=False,
                                             use_tc_tiling_on_sc=False),
        out_type=[
            jax.ShapeDtypeStruct((B, N, U), jnp.float32),
            jax.ShapeDtypeStruct((B, 1, NP), jnp.float32),
        ],
        scratch_types=[
            pltpu.VMEM_SHARED((N, U), jnp.float32),
            pltpu.VMEM_SHARED((N + PAD_ROWS, U), jnp.float32),
            pltpu.VMEM_SHARED((NP,), jnp.float32),
            pltpu.VMEM((NP,), jnp.float32),
            pltpu.VMEM((NP,), jnp.float32),
            pltpu.VMEM((1, 16), jnp.float32),
            pltpu.VMEM((3, W), jnp.int32),
            pltpu.VMEM((3, W), jnp.int32),
            pltpu.VMEM((3, W, U), jnp.float32),
            pltpu.VMEM((3, W), jnp.float32),
            pltpu.VMEM((ZC, U), jnp.float32),
            pltpu.VMEM((RC,), jnp.float32),
            pltpu.SemaphoreType.DMA,
            pltpu.SemaphoreType.DMA,
            pltpu.SemaphoreType.DMA,
            pltpu.SemaphoreType.DMA,
            pltpu.SemaphoreType.DMA,
            pltpu.SemaphoreType.DMA,
            pltpu.SemaphoreType.DMA,
            pltpu.SemaphoreType.DMA,
            pltpu.SemaphoreType.DMA,
        ],
    )()
    return f(h, a_s, a_d, amax, src, dst)


def kernel(x, edge_index, W_gat, att_src, att_dst, b_gat, W1, b1, W2, b2):
    loop = jnp.arange(N, dtype=jnp.int32)
    pad = E_PAD - EP
    src = jnp.concatenate([edge_index[0], loop,
                           jnp.zeros((pad,), jnp.int32)])
    dst = jnp.concatenate([edge_index[1], loop,
                           N + (jnp.arange(pad, dtype=jnp.int32) % PAD_ROWS)])

    wgh = W_gat[:U]
    wgx = W_gat[U:U + 1]
    ats = att_src.reshape(1, U)
    atd = att_dst.reshape(1, U)
    bg = b_gat.reshape(1, U)
    w1x = W1[0:1]
    w1h = W1[1:]
    b1r = b1.reshape(1, 2 * U)
    w2x = W2[0:1]
    w2h = W2[1:]
    b2r = b2.reshape(1, U)

    state = jnp.zeros((B, N, U), dtype=jnp.float32)
    for t in range(T):
        xt = x[:, t]
        h, a_s, a_d, amax = _tc_pre(state, xt, wgh, wgx, ats, atd)
        pad3 = ((0, 0), (0, 0), (0, NP - N))
        num, den = _sc_edge(h,
                            jnp.pad(a_s.reshape(B, 1, N), pad3),
                            jnp.pad(a_d.reshape(B, 1, N), pad3),
                            amax, src, dst)
        state = _tc_post(num, den[:, 0, :N].reshape(B, N, 1), xt, bg,
                         w1x, w1h, b1r, w2x, w2h, b2r)
    return state.reshape(B, N * U)


# speedup vs baseline: 35.3996x; 1.1312x over previous
import functools

import jax
import jax.numpy as jnp
from jax import lax
from jax.experimental import pallas as pl
from jax.experimental.pallas import tpu as pltpu
from jax.experimental.pallas import tpu_sc as plsc

N = 10000
E = 160000
B = 8
T = 4
U = 32

NS = 16
NC = 2
W = 1344
EP = E + N
EPT = ((EP + NS * W - 1) // (NS * W)) * W
E_PAD = EPT * NS
NWIN = EPT // W
PAD_ROWS = 8
RC = 640
RC_LAST = N - RC * (NS - 1)
NP = RC * NS
ZC = 80
BPC = B // NC


def _tc_pre_body(state_ref, xt_ref, wgh_ref, wgx_ref, ats_ref, atd_ref,
                 h_ref, as_ref, ad_ref, amax_ref):
    st = state_ref[0]
    xt = xt_ref[0]
    h = jnp.dot(st, wgh_ref[...], preferred_element_type=jnp.float32)
    h = h + xt * wgx_ref[...]
    h_ref[0] = h
    a_s = jnp.sum(h * ats_ref[...], axis=1, keepdims=True)
    a_d = jnp.sum(h * atd_ref[...], axis=1, keepdims=True)
    as_ref[0] = a_s
    ad_ref[0] = a_d
    amax_ref[0] = jnp.full((1, 16), jnp.max(a_s), dtype=jnp.float32)


def _tc_pre(state, xt, wgh, wgx, ats, atd):
    return pl.pallas_call(
        _tc_pre_body,
        grid=(B,),
        in_specs=[
            pl.BlockSpec((1, N, U), lambda b: (b, 0, 0)),
            pl.BlockSpec((1, N, 1), lambda b: (b, 0, 0)),
            pl.BlockSpec((U, U), lambda b: (0, 0)),
            pl.BlockSpec((1, U), lambda b: (0, 0)),
            pl.BlockSpec((1, U), lambda b: (0, 0)),
            pl.BlockSpec((1, U), lambda b: (0, 0)),
        ],
        out_specs=[
            pl.BlockSpec((1, N, U), lambda b: (b, 0, 0)),
            pl.BlockSpec((1, N, 1), lambda b: (b, 0, 0)),
            pl.BlockSpec((1, N, 1), lambda b: (b, 0, 0)),
            pl.BlockSpec((1, 1, 16), lambda b: (b, 0, 0)),
        ],
        out_shape=[
            jax.ShapeDtypeStruct((B, N, U), jnp.float32),
            jax.ShapeDtypeStruct((B, N, 1), jnp.float32),
            jax.ShapeDtypeStruct((B, N, 1), jnp.float32),
            jax.ShapeDtypeStruct((B, 1, 16), jnp.float32),
        ],
    )(state, xt, wgh, wgx, ats, atd)


def _tc_post_body(num_ref, den_ref, xt_ref, bg_ref,
                  w1x_ref, w1h_ref, b1_ref, w2x_ref, w2h_ref, b2_ref,
                  out_ref):
    num = num_ref[0]
    den = den_ref[0]
    xt = xt_ref[0]
    s2 = num / jnp.maximum(den, 1e-30) + bg_ref[...]
    v = xt * w1x_ref[...] + jnp.dot(s2, w1h_ref[...],
                                    preferred_element_type=jnp.float32)
    v = jax.nn.sigmoid(v + b1_ref[...])
    r = v[:, :U]
    u = v[:, U:]
    c = xt * w2x_ref[...] + jnp.dot(r * s2, w2h_ref[...],
                                    preferred_element_type=jnp.float32)
    c = jnp.tanh(c + b2_ref[...])
    out_ref[0] = u * s2 + (1.0 - u) * c


def _tc_post(num, den, xt, bg, w1x, w1h, b1, w2x, w2h, b2):
    return pl.pallas_call(
        _tc_post_body,
        grid=(B,),
        in_specs=[
            pl.BlockSpec((1, N, U), lambda b: (b, 0, 0)),
            pl.BlockSpec((1, N, 1), lambda b: (b, 0, 0)),
            pl.BlockSpec((1, N, 1), lambda b: (b, 0, 0)),
            pl.BlockSpec((1, U), lambda b: (0, 0)),
            pl.BlockSpec((1, 2 * U), lambda b: (0, 0)),
            pl.BlockSpec((U, 2 * U), lambda b: (0, 0)),
            pl.BlockSpec((1, 2 * U), lambda b: (0, 0)),
            pl.BlockSpec((1, U), lambda b: (0, 0)),
            pl.BlockSpec((U, U), lambda b: (0, 0)),
            pl.BlockSpec((1, U), lambda b: (0, 0)),
        ],
        out_specs=pl.BlockSpec((1, N, U), lambda b: (b, 0, 0)),
        out_shape=jax.ShapeDtypeStruct((B, N, U), jnp.float32),
    )(num, den, xt, bg, w1x, w1h, b1, w2x, w2h, b2)


def _sc_edge_body(h_hbm, as_hbm, ad_hbm, amax_hbm, src_hbm, dst_hbm,
                  num_out, den_out,
                  h_sp, tnum_sp, tden_sp,
                  as_t, ad_t, amax_t, src_w, dst_w, rows, ev_buf,
                  znum, zden,
                  semi0, semi1, semi2, semg0, semg1, semg2,
                  sems0, sems1, sems2):
    sem_i = [semi0, semi1, semi2]
    sem_g = [semg0, semg1, semg2]
    sem_s = [sems0, sems1, sems2]
    c = lax.axis_index("c")
    s = lax.axis_index("s")
    zero16 = jnp.zeros((16,), jnp.float32)
    e16 = lax.iota(jnp.int32, 16)

    def _zn(i, carry):
        znum[i, pl.ds(0, 16)] = zero16
        znum[i, pl.ds(16, 16)] = zero16
        return carry
    lax.fori_loop(0, ZC, _zn, 0)

    def _zd(i, carry):
        zden[pl.ds(i * 16, 16)] = zero16
        return carry
    lax.fori_loop(0, RC // 16, _zd, 0)

    row0 = s * RC
    is_last = s == NS - 1

    def batch_body(i, carry):
        b = c * BPC + i
        def stage_rows(sz):
            def f():
                for i in range(sz // ZC):
                    pltpu.sync_copy(znum,
                                    tnum_sp.at[pl.ds(row0 + i * ZC, ZC)])
                pltpu.sync_copy(h_hbm.at[b, pl.ds(row0, sz)],
                                h_sp.at[pl.ds(row0, sz)])
            return f
        pl.when(is_last)(stage_rows(RC_LAST))
        pl.when(jnp.logical_not(is_last))(stage_rows(RC))
        pltpu.sync_copy(zden, tden_sp.at[pl.ds(row0, RC)])
        pltpu.sync_copy(as_hbm.at[b, 0], as_t)
        pltpu.sync_copy(ad_hbm.at[b, 0], ad_t)
        pltpu.sync_copy(amax_hbm.at[b], amax_t)
        plsc.subcore_barrier()

        amax_v = amax_t[0]

        tb = s * EPT

        def compute_win(p):
            rw = rows.at[p]
            def j_body(j, carry3):
                jb = j * 16
                sv = src_w[p, pl.ds(jb, 16)]
                dv = dst_w[p, pl.ds(jb, 16)]
                asv = plsc.load_gather(as_t, [sv])
                adv = plsc.load_gather(ad_t, [dv])
                al = asv + adv
                al = jnp.where(al >= 0.0, al, al * 0.2)
                mv = amax_v + adv
                mv = jnp.where(mv >= 0.0, mv, mv * 0.2)
                ev = jnp.exp(al - mv)
                ev_buf[p, pl.ds(jb, 16)] = ev
                for i in range(16):
                    evv = jnp.full((16,), ev[i], jnp.float32)
                    r0 = rw[jb + i, pl.ds(0, 16)]
                    rw[jb + i, pl.ds(0, 16)] = r0 * evv
                    r1 = rw[jb + i, pl.ds(16, 16)]
                    rw[jb + i, pl.ds(16, 16)] = r1 * evv
                return carry3
            lax.fori_loop(0, W // 16, j_body, 0)

        def win_body(w, carry2):
            base = tb + w * W
            p = 0
            pltpu.sync_copy(src_hbm.at[pl.ds(base, W)], src_w.at[p])
            pltpu.sync_copy(dst_hbm.at[pl.ds(base, W)], dst_w.at[p])
            pltpu.sync_copy(h_sp.at[src_w.at[p]], rows.at[p])
            compute_win(p)
            pltpu.sync_copy(rows.at[p], tnum_sp.at[dst_w.at[p]], add=True)
            pltpu.sync_copy(ev_buf.at[p], tden_sp.at[dst_w.at[p]], add=True)
            return carry2
        lax.fori_loop(0, NWIN, win_body, 0)
        plsc.subcore_barrier()

        def readout(sz):
            def f():
                pltpu.sync_copy(tnum_sp.at[pl.ds(row0, sz)],
                                num_out.at[b, pl.ds(row0, sz)])
            return f
        pl.when(is_last)(readout(RC_LAST))
        pl.when(jnp.logical_not(is_last))(readout(RC))
        pltpu.sync_copy(tden_sp.at[pl.ds(row0, RC)],
                        den_out.at[b, 0, pl.ds(row0, RC)])
        plsc.subcore_barrier()
        return carry
    lax.fori_loop(0, BPC, batch_body, 0)


def _sc_edge(h, a_s, a_d, amax, src, dst):
    mesh = plsc.VectorSubcoreMesh(core_axis_name="c", subcore_axis_name="s")
    f = functools.partial(
        pl.kernel, _sc_edge_body, mesh=mesh,
        compiler_params=pltpu.CompilerParams(needs_layout_passes=False,
                                             use_tc_tiling_on_sc=False),
        out_type=[
            jax.ShapeDtypeStruct((B, N, U), jnp.float32),
            jax.ShapeDtypeStruct((B, 1, NP), jnp.float32),
        ],
        scratch_types=[
            pltpu.VMEM_SHARED((N, U), jnp.float32),
            pltpu.VMEM_SHARED((N + PAD_ROWS, U), jnp.float32),
            pltpu.VMEM_SHARED((NP,), jnp.float32),
            pltpu.VMEM((NP,), jnp.float32),
            pltpu.VMEM((NP,), jnp.float32),
            pltpu.VMEM((1, 16), jnp.float32),
            pltpu.VMEM((1, W), jnp.int32),
            pltpu.VMEM((1, W), jnp.int32),
            pltpu.VMEM((1, W, U), jnp.float32),
            pltpu.VMEM((1, W), jnp.float32),
            pltpu.VMEM((ZC, U), jnp.float32),
            pltpu.VMEM((RC,), jnp.float32),
            pltpu.SemaphoreType.DMA,
            pltpu.SemaphoreType.DMA,
            pltpu.SemaphoreType.DMA,
            pltpu.SemaphoreType.DMA,
            pltpu.SemaphoreType.DMA,
            pltpu.SemaphoreType.DMA,
            pltpu.SemaphoreType.DMA,
            pltpu.SemaphoreType.DMA,
            pltpu.SemaphoreType.DMA,
        ],
    )()
    return f(h, a_s, a_d, amax, src, dst)


def kernel(x, edge_index, W_gat, att_src, att_dst, b_gat, W1, b1, W2, b2):
    loop = jnp.arange(N, dtype=jnp.int32)
    pad = E_PAD - EP
    src = jnp.concatenate([edge_index[0], loop,
                           jnp.zeros((pad,), jnp.int32)])
    dst = jnp.concatenate([edge_index[1], loop,
                           N + (jnp.arange(pad, dtype=jnp.int32) % PAD_ROWS)])

    wgh = W_gat[:U]
    wgx = W_gat[U:U + 1]
    ats = att_src.reshape(1, U)
    atd = att_dst.reshape(1, U)
    bg = b_gat.reshape(1, U)
    w1x = W1[0:1]
    w1h = W1[1:]
    b1r = b1.reshape(1, 2 * U)
    w2x = W2[0:1]
    w2h = W2[1:]
    b2r = b2.reshape(1, U)

    state = jnp.zeros((B, N, U), dtype=jnp.float32)
    for t in range(T):
        xt = x[:, t]
        h, a_s, a_d, amax = _tc_pre(state, xt, wgh, wgx, ats, atd)
        pad3 = ((0, 0), (0, 0), (0, NP - N))
        num, den = _sc_edge(h,
                            jnp.pad(a_s.reshape(B, 1, N), pad3),
                            jnp.pad(a_d.reshape(B, 1, N), pad3),
                            amax, src, dst)
        state = _tc_post(num, den[:, 0, :N].reshape(B, N, 1), xt, bg,
                         w1x, w1h, b1r, w2x, w2h, b2r)
    return state.reshape(B, N * U)
